# baseline (speedup 1.0000x reference)
import functools

import jax
import jax.numpy as jnp
from jax import lax
from jax.experimental import pallas as pl
from jax.experimental.pallas import tpu as pltpu
from jax.experimental.pallas import tpu_sc as plsc

N_NODES = 10000
N_EDGES = 320000
HDIM = 128

NC = 2
NS = 16
CHUNK = 125
NBUF = 2
NIDX = 8
NCHUNKS = 80
GROUP = 8
ACC_ROWS = 10112
ZROWS = ACC_ROWS // NS


def _sc_segment_sum(h, idx, zeros):
    mesh = plsc.VectorSubcoreMesh(core_axis_name="c", subcore_axis_name="s")

    @functools.partial(
        pl.kernel,
        out_type=jax.ShapeDtypeStruct((NC, ACC_ROWS, HDIM), jnp.float32),
        mesh=mesh,
        scratch_types=[
            pltpu.VMEM((NIDX, 2, CHUNK), jnp.int32),
            pltpu.VMEM((NBUF, CHUNK, HDIM), jnp.float32),
            pltpu.VMEM_SHARED((ACC_ROWS, HDIM), jnp.float32),
            pltpu.SemaphoreType.DMA((NIDX, 2)),
            pltpu.SemaphoreType.DMA((NBUF,)),
            pltpu.SemaphoreType.DMA((NBUF,)),
        ],
    )
    def k(h_hbm, idx_hbm, zero_hbm, out_hbm, idxr, rows_v, acc, isem, gsem, ssem):
        cid = lax.axis_index("c")
        sid = lax.axis_index("s")

        pltpu.sync_copy(zero_hbm.at[pl.ds(sid * ZROWS, ZROWS)],
                        acc.at[pl.ds(sid * ZROWS, ZROWS)])
        plsc.subcore_barrier()

        def fetch_idx(g, s):
            for p in range(2):
                pltpu.async_copy(idx_hbm.at[p, cid, sid, g], idxr.at[s, p],
                                 isem.at[s, p])

        def wait_idx(s):
            for p in range(2):
                pltpu.make_async_copy(idx_hbm.at[p, cid, sid, 0], idxr.at[s, p],
                                      isem.at[s, p]).wait()

        for c in range(5):
            fetch_idx(c, c)
        wait_idx(0)
        pltpu.async_copy(h_hbm.at[idxr.at[0, 0]], rows_v.at[0], gsem.at[0])

        def group(G2, carry):
            for u in range(GROUP):
                g = G2 * GROUP + u
                b = u % NBUF
                s = u % NIDX
                s1 = (u + 1) % NIDX
                s5 = (u + 5) % NIDX
                pltpu.make_async_copy(h_hbm.at[idxr.at[s, 0]], rows_v.at[b],
                                      gsem.at[b]).wait()
                pltpu.async_copy(rows_v.at[b], acc.at[idxr.at[s, 1]],
                                 ssem.at[b], add=True)

                @pl.when(g >= 1)
                def _():
                    pltpu.make_async_copy(rows_v.at[1 - b],
                                          acc.at[idxr.at[s1, 1]],
                                          ssem.at[1 - b]).wait()

                @pl.when(g + 1 < NCHUNKS)
                def _():
                    wait_idx(s1)
                    pltpu.async_copy(h_hbm.at[idxr.at[s1, 0]], rows_v.at[1 - b],
                                     gsem.at[1 - b])

                @pl.when(g + 5 < NCHUNKS)
                def _():
                    fetch_idx(g + 5, s5)
            return carry

        lax.fori_loop(0, NCHUNKS // GROUP, group, 0, unroll=False)

        lb = (NCHUNKS - 1) % NBUF
        pltpu.make_async_copy(rows_v.at[lb],
                              acc.at[idxr.at[(NCHUNKS - 1) % NIDX, 1]],
                              ssem.at[lb]).wait()

        plsc.subcore_barrier()
        pltpu.sync_copy(acc.at[pl.ds(sid * ZROWS, ZROWS)],
                        out_hbm.at[cid, pl.ds(sid * ZROWS, ZROWS)])

    return k(h, idx, zeros)


def _fused_body(x_ref, m_ref, w_ref, b_ref, p0_ref, p1_ref, u_ref, o_ref):
    xwb = jnp.dot(x_ref[...], w_ref[...], preferred_element_type=jnp.float32)
    xwb = (xwb + b_ref[...]) * m_ref[...]
    hsum = p0_ref[0] + p1_ref[0]
    h_aggr = jnp.dot(hsum, u_ref[...], preferred_element_type=jnp.float32)
    o_ref[...] = jnp.tanh(xwb + h_aggr)


def _fused_stage(x, mask2d, W_in, b2d, partials, U):
    R = 1000
    return pl.pallas_call(
        _fused_body,
        grid=(N_NODES // R,),
        in_specs=[
            pl.BlockSpec((R, HDIM), lambda i: (i, 0)),
            pl.BlockSpec((R, 1), lambda i: (i, 0)),
            pl.BlockSpec((HDIM, HDIM), lambda i: (0, 0)),
            pl.BlockSpec((1, HDIM), lambda i: (0, 0)),
            pl.BlockSpec((1, R, HDIM), lambda i: (0, i, 0)),
            pl.BlockSpec((1, R, HDIM), lambda i: (1, i, 0)),
            pl.BlockSpec((HDIM, HDIM), lambda i: (0, 0)),
        ],
        out_specs=pl.BlockSpec((R, HDIM), lambda i: (i, 0)),
        out_shape=jax.ShapeDtypeStruct((N_NODES, HDIM), jnp.float32),
    )(x, mask2d, W_in, b2d, partials, partials, U)


def kernel(x, x_mask, h, edge_index, W_in, b_in, U):
    idx = edge_index.astype(jnp.int32).reshape(2, NC, NS, NCHUNKS, CHUNK)
    zeros = jnp.zeros((ACC_ROWS, HDIM), jnp.float32)

    partials = _sc_segment_sum(h, idx, zeros)

    mask2d = x_mask.reshape(N_NODES, 1)
    b2d = b_in.reshape(1, HDIM)
    return _fused_stage(x, mask2d, W_in, b2d, partials, U)
